# baseline (speedup 1.0000x reference)
import math

import jax
import jax.numpy as jnp
from jax.experimental import pallas as pl

N = 10000
E = 320000
D_IN = 128
D_MSG = 64
D_OUT = 128


def _pre_body(x_ref, w1t_ref, y_ref, s_ref):
    x = x_ref[...]
    y_ref[...] = jnp.dot(x, w1t_ref[...], preferred_element_type=jnp.float32)
    s_ref[...] = jnp.sum(x, axis=1) * (1.0 / math.sqrt(D_IN))


_pre = pl.pallas_call(
    _pre_body,
    out_shape=(
        jax.ShapeDtypeStruct((N, D_MSG), jnp.float32),
        jax.ShapeDtypeStruct((N,), jnp.float32),
    ),
)


def _mid_body(agg_ref, z_ref, y_ref, h_ref, s2_ref):
    inv = (1.0 / (z_ref[...] + 1.0))[:, None]
    h = jnp.maximum(agg_ref[...] + y_ref[...] * inv, 0.0)
    h_ref[...] = h
    s2_ref[...] = jnp.sum(h, axis=1) * (1.0 / math.sqrt(D_MSG))


_mid = pl.pallas_call(
    _mid_body,
    out_shape=(
        jax.ShapeDtypeStruct((N, D_MSG), jnp.float32),
        jax.ShapeDtypeStruct((N,), jnp.float32),
    ),
)


def _post_body(agg_ref, z_ref, h_ref, w2t_ref, out_ref):
    inv = (1.0 / (z_ref[...] + 1.0))[:, None]
    a = agg_ref[...] + h_ref[...] * inv
    o = jnp.dot(a, w2t_ref[...], preferred_element_type=jnp.float32)
    m = jnp.max(o, axis=1, keepdims=True)
    lse = jnp.log(jnp.sum(jnp.exp(o - m), axis=1, keepdims=True)) + m
    out_ref[...] = o - lse


_post = pl.pallas_call(
    _post_body,
    out_shape=jax.ShapeDtypeStruct((N, D_OUT), jnp.float32),
)


def _edge_aggregate(s, y, src, dst):
    a = jnp.take(s, dst, axis=0)
    b = jnp.take(s, src, axis=0)
    u = jnp.exp(4.5 * (jnp.abs(a + b) - jnp.abs(a - b)) - 9.0 * jnp.abs(b))
    z = jax.ops.segment_sum(u, src, num_segments=N)
    w = u / (jnp.take(z, src, axis=0) + 1.0)
    agg = jax.ops.segment_sum(w[:, None] * jnp.take(y, src, axis=0), dst,
                              num_segments=N)
    return z, agg


def kernel(x, edge_index, W1, W2):
    src = edge_index[0]
    dst = edge_index[1]
    y, s1 = _pre(x, W1.T)
    z1, agg1 = _edge_aggregate(s1, y, src, dst)
    h, s2 = _mid(agg1, z1, y)
    z2, agg2 = _edge_aggregate(s2, h, src, dst)
    return _post(agg2, z2, h, W2.T)


# speedup vs baseline: 2.4439x; 2.4439x over previous
import math

import jax
import jax.numpy as jnp
from jax import lax
from jax.experimental import pallas as pl
from jax.experimental.pallas import tpu as pltpu
from jax.experimental.pallas import tpu_sc as plsc

N = 10000
E = 320000
D_IN = 128
D_MSG = 64
D_OUT = 128

NC = 2
NS = 16
NW = NC * NS
L = 16
NPAD = 10240
ZR = NPAD // 128
EPAD = 327680
E_W = EPAD // NW
ER = E_W // 128


def _sc_z_body(s_hbm, src_hbm, dst_hbm, zp_hbm, u_hbm,
               s_loc, srcz, dstz, zbuf, ubuf):
    c = lax.axis_index("c")
    t = lax.axis_index("s")

    pltpu.sync_copy(s_hbm, s_loc)

    z16 = jnp.zeros((L,), jnp.float32)

    @pl.loop(0, ZR)
    def _(i):
        for q in range(128 // L):
            zbuf[i, pl.ds(q * L, L)] = z16

    for cc in range(NC):
        for k in range(NS):
            w = cc * NS + k

            @pl.when(jnp.logical_and(c == cc, t == k))
            def _(w=w):
                pltpu.sync_copy(src_hbm.at[w], srcz)
                pltpu.sync_copy(dst_hbm.at[w], dstz)

    @pl.loop(0, E_W // L)
    def _(j):
        jr = lax.shift_right_logical(j, 3)
        jc = jnp.bitwise_and(j, 7) * L
        sl = pl.ds(jc, L)
        sv = srcz[jr, sl]
        dv = dstz[jr, sl]
        b = plsc.load_gather(s_loc, [sv])
        a = plsc.load_gather(s_loc, [dv])
        u = jnp.exp(4.5 * (jnp.abs(a + b) - jnp.abs(a - b)) - 9.0 * jnp.abs(b))
        ubuf[jr, sl] = u
        zr = lax.shift_right_logical(sv, 7)
        zc = jnp.bitwise_and(sv, 127)
        plsc.addupdate_scatter(zbuf, [zr, zc], u)

    for cc in range(NC):
        for k in range(NS):
            w = cc * NS + k

            @pl.when(jnp.logical_and(c == cc, t == k))
            def _(w=w):
                pltpu.sync_copy(zbuf, zp_hbm.at[w])
                pltpu.sync_copy(ubuf, u_hbm.at[w])


_sc_z = pl.kernel(
    _sc_z_body,
    out_type=(
        jax.ShapeDtypeStruct((NW, ZR, 128), jnp.float32),
        jax.ShapeDtypeStruct((NW, ER, 128), jnp.float32),
    ),
    mesh=plsc.VectorSubcoreMesh(core_axis_name="c", subcore_axis_name="s"),
    compiler_params=pltpu.CompilerParams(needs_layout_passes=False),
    scratch_types=[
        pltpu.VMEM((NPAD,), jnp.float32),
        pltpu.VMEM((ER, 128), jnp.int32),
        pltpu.VMEM((ER, 128), jnp.int32),
        pltpu.VMEM((ZR, 128), jnp.float32),
        pltpu.VMEM((ER, 128), jnp.float32),
    ],
)


def _pre_body(x_ref, w1t_ref, ei_ref, y_ref, s_ref, src_ref, dst_ref):
    x = x_ref[...]
    y_ref[...] = jnp.dot(x, w1t_ref[...], preferred_element_type=jnp.float32)
    s = jnp.sum(x, axis=1) * (1.0 / math.sqrt(D_IN))
    s_ref[pl.ds(0, N)] = s
    s_ref[pl.ds(N, NPAD - N)] = jnp.zeros((NPAD - N,), jnp.float32)
    src_ref[...] = ei_ref[0].reshape(NW, ER, 128)
    dst_ref[...] = ei_ref[1].reshape(NW, ER, 128)


_pre = pl.pallas_call(
    _pre_body,
    out_shape=(
        jax.ShapeDtypeStruct((N, D_MSG), jnp.float32),
        jax.ShapeDtypeStruct((NPAD,), jnp.float32),
        jax.ShapeDtypeStruct((NW, ER, 128), jnp.int32),
        jax.ShapeDtypeStruct((NW, ER, 128), jnp.int32),
    ),
)


def _mid_body(agg_ref, z_ref, y_ref, h_ref, s2_ref):
    inv = (1.0 / (z_ref[...] + 1.0))[:, None]
    h = jnp.maximum(agg_ref[...] + y_ref[...] * inv, 0.0)
    h_ref[...] = h
    s2 = jnp.sum(h, axis=1) * (1.0 / math.sqrt(D_MSG))
    s2_ref[pl.ds(0, N)] = s2
    s2_ref[pl.ds(N, NPAD - N)] = jnp.zeros((NPAD - N,), jnp.float32)


_mid = pl.pallas_call(
    _mid_body,
    out_shape=(
        jax.ShapeDtypeStruct((N, D_MSG), jnp.float32),
        jax.ShapeDtypeStruct((NPAD,), jnp.float32),
    ),
)


def _zsum_body(zp_ref, z_ref):
    z_ref[...] = jnp.sum(zp_ref[...], axis=0)[:N]


_zsum = pl.pallas_call(
    _zsum_body,
    out_shape=jax.ShapeDtypeStruct((N,), jnp.float32),
)


def _post_body(agg_ref, z_ref, h_ref, w2t_ref, out_ref):
    inv = (1.0 / (z_ref[...] + 1.0))[:, None]
    a = agg_ref[...] + h_ref[...] * inv
    o = jnp.dot(a, w2t_ref[...], preferred_element_type=jnp.float32)
    m = jnp.max(o, axis=1, keepdims=True)
    lse = jnp.log(jnp.sum(jnp.exp(o - m), axis=1, keepdims=True)) + m
    out_ref[...] = o - lse


_post = pl.pallas_call(
    _post_body,
    out_shape=jax.ShapeDtypeStruct((N, D_OUT), jnp.float32),
)


def _edge_layer(s, y, src, dst, src3, dst3):
    zp, up = _sc_z(s, src3, dst3)
    z = _zsum(zp.reshape(NW, NPAD))
    u = up.reshape(EPAD)[:E]
    w = u / (jnp.take(z, src, axis=0) + 1.0)
    agg = jax.ops.segment_sum(w[:, None] * jnp.take(y, src, axis=0), dst,
                              num_segments=N)
    return z, agg


def kernel(x, edge_index, W1, W2):
    src = edge_index[0]
    dst = edge_index[1]
    pad = jnp.full((2, EPAD - E), NPAD - 1, dtype=edge_index.dtype)
    ei = jnp.concatenate([edge_index, pad], axis=1)
    y, s1, src3, dst3 = _pre(x, W1.T, ei)
    z1, agg1 = _edge_layer(s1, y, src, dst, src3, dst3)
    h, s2 = _mid(agg1, z1, y)
    z2, agg2 = _edge_layer(s2, h, src, dst, src3, dst3)
    return _post(agg2, z2, h, W2.T)
